# Initial kernel scaffold; baseline (speedup 1.0000x reference)
import jax
import jax.numpy as jnp
from jax.experimental import pallas as pl


def kernel(x, edge_index, W_rel1, b_rel1, W_root1, W_l1, b_l1, W_rel2, b_rel2, W_root2, W_l2, b_l2):
    raise NotImplementedError("write your pallas kernel here")


# speedup vs baseline: 9.9201x; 9.9201x over previous
import functools

import jax
import jax.numpy as jnp
from jax import lax
from jax.experimental import pallas as pl
from jax.experimental.pallas import tpu as pltpu
from jax.experimental.pallas import tpu_sc as plsc

N = 10000
E = 320000
NC = 2
NS = 16
NW = NC * NS
CH = 80
NCHUNK = E // (NW * CH)
NP = 10240
ROWS_PER_SUB = NP // NS


def _sc_segment_sum(y, src_r, dst_r, zeros):
    D = y.shape[1]
    mesh = plsc.VectorSubcoreMesh(core_axis_name="c", subcore_axis_name="s")

    @functools.partial(
        pl.kernel,
        out_type=jax.ShapeDtypeStruct((NC, NP, D), jnp.float32),
        mesh=mesh,
        scratch_types=[
            pltpu.VMEM((NCHUNK, CH), jnp.int32),
            pltpu.VMEM((NCHUNK, CH), jnp.int32),
            pltpu.VMEM((CH, D), jnp.float32),
            pltpu.VMEM_SHARED((NP, D), jnp.float32),
            pltpu.SemaphoreType.DMA,
        ],
        compiler_params=pltpu.CompilerParams(use_tc_tiling_on_sc=False),
    )
    def seg_kernel(y_hbm, src_hbm, dst_hbm, zero_hbm, out_hbm,
                   src_v, dst_v, rows_v, acc_sh, sem):
        c = lax.axis_index("c")
        s = lax.axis_index("s")
        w = c * NS + s
        pltpu.sync_copy(zero_hbm.at[pl.ds(s * ROWS_PER_SUB, ROWS_PER_SUB)],
                        acc_sh.at[pl.ds(s * ROWS_PER_SUB, ROWS_PER_SUB)])
        pltpu.sync_copy(src_hbm.at[w], src_v)
        pltpu.sync_copy(dst_hbm.at[w], dst_v)
        plsc.subcore_barrier()

        @pl.loop(0, NCHUNK)
        def _(j):
            pltpu.async_copy(y_hbm.at[src_v.at[j]], rows_v, sem).wait()
            pltpu.sync_copy(rows_v, acc_sh.at[dst_v.at[j]], add=True)

        plsc.subcore_barrier()
        pltpu.sync_copy(acc_sh.at[pl.ds(s * ROWS_PER_SUB, ROWS_PER_SUB)],
                        out_hbm.at[c, pl.ds(s * ROWS_PER_SUB, ROWS_PER_SUB)])

    return seg_kernel(y, src_r, dst_r, zeros)


def _dot_t(a, w):
    return lax.dot_general(a, w, (((1,), (1,)), ((), ())),
                           preferred_element_type=jnp.float32)


def _tc_pre(x, W_rel1, W_root1):
    def body(x_ref, wr_ref, wo_ref, y_ref, xr_ref):
        xv = x_ref[...]
        y_ref[...] = _dot_t(xv, wr_ref[...])
        xr_ref[...] = _dot_t(xv, wo_ref[...])

    return pl.pallas_call(
        body,
        out_shape=[jax.ShapeDtypeStruct((N, W_rel1.shape[0]), jnp.float32),
                   jax.ShapeDtypeStruct((N, W_root1.shape[0]), jnp.float32)],
    )(x, W_rel1, W_root1)


def _tc_mid(part1, xr1, b_rel1, W_l1, b_l1, W_rel2, W_root2, b_rel2):
    def body(p_ref, xr_ref, br1_ref, wl1_ref, bl1_ref, wr2_ref, wo2_ref,
             br2_ref, y2_ref, hr2_ref):
        h1 = p_ref[0, :N] + p_ref[1, :N] + xr_ref[...] + br1_ref[...]
        h2 = jnp.maximum(_dot_t(h1, wl1_ref[...]) + bl1_ref[...], 0.0)
        y2_ref[...] = _dot_t(h2, wr2_ref[...])
        hr2_ref[...] = _dot_t(h2, wo2_ref[...]) + br2_ref[...]

    return pl.pallas_call(
        body,
        out_shape=[jax.ShapeDtypeStruct((N, W_rel2.shape[0]), jnp.float32),
                   jax.ShapeDtypeStruct((N, W_root2.shape[0]), jnp.float32)],
    )(part1, xr1, b_rel1.reshape(1, -1), W_l1, b_l1.reshape(1, -1),
      W_rel2, W_root2, b_rel2.reshape(1, -1))


def _tc_post(part2, hr2, W_l2, b_l2):
    def body(p_ref, hr_ref, wl2_ref, bl2_ref, o_ref):
        h3 = p_ref[0, :N] + p_ref[1, :N] + hr_ref[...]
        logits = _dot_t(h3, wl2_ref[...]) + bl2_ref[...]
        m = jnp.max(logits, axis=1, keepdims=True)
        shifted = logits - m
        lse = jnp.log(jnp.sum(jnp.exp(shifted), axis=1, keepdims=True))
        o_ref[...] = shifted - lse

    return pl.pallas_call(
        body,
        out_shape=jax.ShapeDtypeStruct((N, W_l2.shape[0]), jnp.float32),
    )(part2, hr2, W_l2, b_l2.reshape(1, -1))


def kernel(x, edge_index, W_rel1, b_rel1, W_root1, W_l1, b_l1,
           W_rel2, b_rel2, W_root2, W_l2, b_l2):
    src_r = edge_index[0].reshape(NW, NCHUNK, CH)
    dst_r = edge_index[1].reshape(NW, NCHUNK, CH)

    y1, xr1 = _tc_pre(x, W_rel1, W_root1)
    part1 = _sc_segment_sum(y1, src_r, dst_r,
                            jnp.zeros((NP, W_rel1.shape[0]), jnp.float32))
    y2, hr2 = _tc_mid(part1, xr1, b_rel1, W_l1, b_l1, W_rel2, W_root2, b_rel2)
    part2 = _sc_segment_sum(y2, src_r, dst_r,
                            jnp.zeros((NP, W_rel2.shape[0]), jnp.float32))
    return _tc_post(part2, hr2, W_l2, b_l2)


# speedup vs baseline: 13.5896x; 1.3699x over previous
import functools

import jax
import jax.numpy as jnp
from jax import lax
from jax.experimental import pallas as pl
from jax.experimental.pallas import tpu as pltpu
from jax.experimental.pallas import tpu_sc as plsc

N = 10000
E = 320000
NC = 2
NS = 16
NW = NC * NS
CH = 128
EW = E // NW
NCHUNK = -(-EW // CH)
EWP = NCHUNK * CH
NP = 10240
ROWS_PER_SUB = NP // NS


def _sc_segment_sum(y, src_r, dst_r, zeros):
    D = y.shape[1]
    mesh = plsc.VectorSubcoreMesh(core_axis_name="c", subcore_axis_name="s")

    @functools.partial(
        pl.kernel,
        out_type=jax.ShapeDtypeStruct((NC, NP, D), jnp.float32),
        mesh=mesh,
        scratch_types=[
            pltpu.VMEM((NCHUNK, CH), jnp.int32),
            pltpu.VMEM((NCHUNK, CH), jnp.int32),
            pltpu.VMEM((CH, D), jnp.float32),
            pltpu.VMEM((CH, D), jnp.float32),
            pltpu.VMEM_SHARED((NP, D), jnp.float32),
            pltpu.SemaphoreType.DMA,
            pltpu.SemaphoreType.DMA,
        ],
        compiler_params=pltpu.CompilerParams(use_tc_tiling_on_sc=False),
    )
    def seg_kernel(y_hbm, src_hbm, dst_hbm, zero_hbm, out_hbm,
                   src_v, dst_v, rows_v0, rows_v1, acc_sh, sem0, sem1):
        c = lax.axis_index("c")
        s = lax.axis_index("s")
        w = c * NS + s
        pltpu.sync_copy(zero_hbm.at[pl.ds(s * ROWS_PER_SUB, ROWS_PER_SUB)],
                        acc_sh.at[pl.ds(s * ROWS_PER_SUB, ROWS_PER_SUB)])
        pltpu.sync_copy(src_hbm.at[w], src_v)
        pltpu.sync_copy(dst_hbm.at[w], dst_v)
        plsc.subcore_barrier()

        pltpu.async_copy(y_hbm.at[src_v.at[0]], rows_v0, sem0)

        @pl.loop(0, NCHUNK - 1, step=2)
        def _(j):
            pltpu.make_async_copy(y_hbm.at[src_v.at[j]], rows_v0, sem0).wait()
            pltpu.async_copy(y_hbm.at[src_v.at[j + 1]], rows_v1, sem1)
            pltpu.sync_copy(rows_v0, acc_sh.at[dst_v.at[j]], add=True)
            pltpu.make_async_copy(y_hbm.at[src_v.at[j + 1]], rows_v1,
                                  sem1).wait()
            pltpu.async_copy(y_hbm.at[src_v.at[j + 2]], rows_v0, sem0)
            pltpu.sync_copy(rows_v1, acc_sh.at[dst_v.at[j + 1]], add=True)

        pltpu.make_async_copy(y_hbm.at[src_v.at[NCHUNK - 1]], rows_v0,
                              sem0).wait()
        pltpu.sync_copy(rows_v0, acc_sh.at[dst_v.at[NCHUNK - 1]], add=True)

        plsc.subcore_barrier()
        pltpu.sync_copy(acc_sh.at[pl.ds(s * ROWS_PER_SUB, ROWS_PER_SUB)],
                        out_hbm.at[c, pl.ds(s * ROWS_PER_SUB, ROWS_PER_SUB)])

    return seg_kernel(y, src_r, dst_r, zeros)


def _dot_t(a, w):
    return lax.dot_general(a, w, (((1,), (1,)), ((), ())),
                           preferred_element_type=jnp.float32)


def _tc_pre(x, W_rel1, W_root1):
    def body(x_ref, wr_ref, wo_ref, y_ref, xr_ref):
        xv = x_ref[...]
        y_ref[...] = _dot_t(xv, wr_ref[...])
        xr_ref[...] = _dot_t(xv, wo_ref[...])

    return pl.pallas_call(
        body,
        out_shape=[jax.ShapeDtypeStruct((N, W_rel1.shape[0]), jnp.float32),
                   jax.ShapeDtypeStruct((N, W_root1.shape[0]), jnp.float32)],
    )(x, W_rel1, W_root1)


def _tc_mid(part1, xr1, b_rel1, W_l1, b_l1, W_rel2, W_root2, b_rel2):
    def body(p_ref, xr_ref, br1_ref, wl1_ref, bl1_ref, wr2_ref, wo2_ref,
             br2_ref, y2_ref, hr2_ref):
        h1 = p_ref[0, :N] + p_ref[1, :N] + xr_ref[...] + br1_ref[...]
        h2 = jnp.maximum(_dot_t(h1, wl1_ref[...]) + bl1_ref[...], 0.0)
        y2_ref[...] = _dot_t(h2, wr2_ref[...])
        hr2_ref[...] = _dot_t(h2, wo2_ref[...]) + br2_ref[...]

    return pl.pallas_call(
        body,
        out_shape=[jax.ShapeDtypeStruct((N, W_rel2.shape[0]), jnp.float32),
                   jax.ShapeDtypeStruct((N, W_root2.shape[0]), jnp.float32)],
    )(part1, xr1, b_rel1.reshape(1, -1), W_l1, b_l1.reshape(1, -1),
      W_rel2, W_root2, b_rel2.reshape(1, -1))


def _tc_post(part2, hr2, W_l2, b_l2):
    def body(p_ref, hr_ref, wl2_ref, bl2_ref, o_ref):
        h3 = p_ref[0, :N] + p_ref[1, :N] + hr_ref[...]
        logits = _dot_t(h3, wl2_ref[...]) + bl2_ref[...]
        m = jnp.max(logits, axis=1, keepdims=True)
        shifted = logits - m
        lse = jnp.log(jnp.sum(jnp.exp(shifted), axis=1, keepdims=True))
        o_ref[...] = shifted - lse

    return pl.pallas_call(
        body,
        out_shape=jax.ShapeDtypeStruct((N, W_l2.shape[0]), jnp.float32),
    )(part2, hr2, W_l2, b_l2.reshape(1, -1))


def kernel(x, edge_index, W_rel1, b_rel1, W_root1, W_l1, b_l1,
           W_rel2, b_rel2, W_root2, W_l2, b_l2):
    npad = EWP - EW
    pad_iota = jnp.arange(NW * npad, dtype=jnp.int32).reshape(NW, npad)
    src_pad = pad_iota % N
    dst_pad = N + (pad_iota % (NP - N))
    src_r = jnp.concatenate(
        [edge_index[0].reshape(NW, EW), src_pad], axis=1).reshape(
            NW, NCHUNK, CH)
    dst_r = jnp.concatenate(
        [edge_index[1].reshape(NW, EW), dst_pad], axis=1).reshape(
            NW, NCHUNK, CH)

    y1, xr1 = _tc_pre(x, W_rel1, W_root1)
    part1 = _sc_segment_sum(y1, src_r, dst_r,
                            jnp.zeros((NP, W_rel1.shape[0]), jnp.float32))
    y2, hr2 = _tc_mid(part1, xr1, b_rel1, W_l1, b_l1, W_rel2, W_root2, b_rel2)
    part2 = _sc_segment_sum(y2, src_r, dst_r,
                            jnp.zeros((NP, W_rel2.shape[0]), jnp.float32))
    return _tc_post(part2, hr2, W_l2, b_l2)


# speedup vs baseline: 19.0552x; 1.4022x over previous
import functools

import jax
import jax.numpy as jnp
from jax import lax
from jax.experimental import pallas as pl
from jax.experimental.pallas import tpu as pltpu
from jax.experimental.pallas import tpu_sc as plsc

N = 10000
E = 320000
NC = 2
NS = 16
NW = NC * NS
CH = 128
EW = E // NW
NCHUNK = 80
EWP = NCHUNK * CH
NP = 10240
ROWS_PER_SUB = NP // NS


def _sc_segment_sum(y, src_r, dst_r, zeros):
    D = y.shape[1]
    mesh = plsc.VectorSubcoreMesh(core_axis_name="c", subcore_axis_name="s")

    @functools.partial(
        pl.kernel,
        out_type=jax.ShapeDtypeStruct((NC, NP, D), jnp.float32),
        mesh=mesh,
        scratch_types=[
            pltpu.VMEM((NCHUNK, CH), jnp.int32),
            pltpu.VMEM((NCHUNK, CH), jnp.int32),
        ] + [pltpu.VMEM((CH, D), jnp.float32) for _ in range(8)] + [
            pltpu.VMEM_SHARED((NP, D), jnp.float32),
            pltpu.SemaphoreType.DMA((8,)),
            pltpu.SemaphoreType.DMA((8,)),
        ],
        compiler_params=pltpu.CompilerParams(use_tc_tiling_on_sc=False),
    )
    def seg_kernel(y_hbm, src_hbm, dst_hbm, zero_hbm, out_hbm,
                   src_v, dst_v, b0, b1, b2, b3, b4, b5, b6, b7,
                   acc_sh, gsem, ssem):
        c = lax.axis_index("c")
        s = lax.axis_index("s")
        w = c * NS + s
        bufs = (b0, b1, b2, b3, b4, b5, b6, b7)

        pltpu.sync_copy(zero_hbm.at[pl.ds(s * ROWS_PER_SUB, ROWS_PER_SUB)],
                        acc_sh.at[pl.ds(s * ROWS_PER_SUB, ROWS_PER_SUB)])
        pltpu.sync_copy(src_hbm.at[w], src_v)
        pltpu.sync_copy(dst_hbm.at[w], dst_v)
        plsc.subcore_barrier()

        def g_start(ci, k):
            pltpu.async_copy(y_hbm.at[src_v.at[ci]], bufs[k], gsem.at[k])

        def g_wait(ci, k):
            pltpu.make_async_copy(y_hbm.at[src_v.at[ci]], bufs[k],
                                  gsem.at[k]).wait()

        def s_start(ci, k):
            pltpu.async_copy(bufs[k], acc_sh.at[dst_v.at[ci]], ssem.at[k],
                             add=True)

        def s_wait(ci, k):
            pltpu.make_async_copy(bufs[k], acc_sh.at[dst_v.at[ci]],
                                  ssem.at[k]).wait()

        for k in range(4):
            g_start(k, k)
        for k in range(8):
            g_wait(k, k)
            s_start(k, k)
            if k >= 4:
                s_wait(k - 4, (k + 4) % 8)
            g_start(k + 4, (k + 4) % 8)

        @pl.loop(8, NCHUNK - 8, step=8)
        def _(j):
            for k in range(8):
                ci = j + k
                g_wait(ci, k)
                s_start(ci, k)
                s_wait(ci - 4, (k + 4) % 8)
                g_start(ci + 4, (k + 4) % 8)

        for k in range(8):
            ci = NCHUNK - 8 + k
            g_wait(ci, k)
            s_start(ci, k)
            s_wait(ci - 4, (k + 4) % 8)
            if k < 4:
                g_start(ci + 4, (k + 4) % 8)
        for k in range(4, 8):
            s_wait(NCHUNK - 8 + k, k)

        plsc.subcore_barrier()
        pltpu.sync_copy(acc_sh.at[pl.ds(s * ROWS_PER_SUB, ROWS_PER_SUB)],
                        out_hbm.at[c, pl.ds(s * ROWS_PER_SUB, ROWS_PER_SUB)])

    return seg_kernel(y, src_r, dst_r, zeros)


def _dot_t(a, w):
    return lax.dot_general(a, w, (((1,), (1,)), ((), ())),
                           preferred_element_type=jnp.float32)


def _tc_pre(x, W_rel1, W_root1):
    def body(x_ref, wr_ref, wo_ref, y_ref, xr_ref):
        xv = x_ref[...]
        y_ref[...] = _dot_t(xv, wr_ref[...])
        xr_ref[...] = _dot_t(xv, wo_ref[...])

    return pl.pallas_call(
        body,
        out_shape=[jax.ShapeDtypeStruct((N, W_rel1.shape[0]), jnp.float32),
                   jax.ShapeDtypeStruct((N, W_root1.shape[0]), jnp.float32)],
    )(x, W_rel1, W_root1)


def _tc_mid(part1, xr1, b_rel1, W_l1, b_l1, W_rel2, W_root2, b_rel2):
    def body(p_ref, xr_ref, br1_ref, wl1_ref, bl1_ref, wr2_ref, wo2_ref,
             br2_ref, y2_ref, hr2_ref):
        h1 = p_ref[0, :N] + p_ref[1, :N] + xr_ref[...] + br1_ref[...]
        h2 = jnp.maximum(_dot_t(h1, wl1_ref[...]) + bl1_ref[...], 0.0)
        y2_ref[...] = _dot_t(h2, wr2_ref[...])
        hr2_ref[...] = _dot_t(h2, wo2_ref[...]) + br2_ref[...]

    return pl.pallas_call(
        body,
        out_shape=[jax.ShapeDtypeStruct((N, W_rel2.shape[0]), jnp.float32),
                   jax.ShapeDtypeStruct((N, W_root2.shape[0]), jnp.float32)],
    )(part1, xr1, b_rel1.reshape(1, -1), W_l1, b_l1.reshape(1, -1),
      W_rel2, W_root2, b_rel2.reshape(1, -1))


def _tc_post(part2, hr2, W_l2, b_l2):
    def body(p_ref, hr_ref, wl2_ref, bl2_ref, o_ref):
        h3 = p_ref[0, :N] + p_ref[1, :N] + hr_ref[...]
        logits = _dot_t(h3, wl2_ref[...]) + bl2_ref[...]
        m = jnp.max(logits, axis=1, keepdims=True)
        shifted = logits - m
        lse = jnp.log(jnp.sum(jnp.exp(shifted), axis=1, keepdims=True))
        o_ref[...] = shifted - lse

    return pl.pallas_call(
        body,
        out_shape=jax.ShapeDtypeStruct((N, W_l2.shape[0]), jnp.float32),
    )(part2, hr2, W_l2, b_l2.reshape(1, -1))


def kernel(x, edge_index, W_rel1, b_rel1, W_root1, W_l1, b_l1,
           W_rel2, b_rel2, W_root2, W_l2, b_l2):
    npad = EWP - EW
    pad_iota = jnp.arange(NW * npad, dtype=jnp.int32).reshape(NW, npad)
    src_pad = pad_iota % N
    dst_pad = N + (pad_iota % (NP - N))
    src_r = jnp.concatenate(
        [edge_index[0].reshape(NW, EW), src_pad], axis=1).reshape(
            NW, NCHUNK, CH)
    dst_r = jnp.concatenate(
        [edge_index[1].reshape(NW, EW), dst_pad], axis=1).reshape(
            NW, NCHUNK, CH)

    y1, xr1 = _tc_pre(x, W_rel1, W_root1)
    part1 = _sc_segment_sum(y1, src_r, dst_r,
                            jnp.zeros((NP, W_rel1.shape[0]), jnp.float32))
    y2, hr2 = _tc_mid(part1, xr1, b_rel1, W_l1, b_l1, W_rel2, W_root2, b_rel2)
    part2 = _sc_segment_sum(y2, src_r, dst_r,
                            jnp.zeros((NP, W_rel2.shape[0]), jnp.float32))
    return _tc_post(part2, hr2, W_l2, b_l2)
